# baseline (speedup 1.0000x reference)
import functools

import jax
import jax.numpy as jnp
from jax import lax
from jax.experimental import pallas as pl
from jax.experimental.pallas import tpu as pltpu
from jax.experimental.pallas import tpu_sc as plsc

N_NODES = 10000
N_EDGES = 320000
D_NODE = 128
D_EDGE = 16
PACK = 128 // D_EDGE
N_PACKED = N_EDGES // PACK

GROUP = 128
N_GROUPS = N_EDGES // GROUP
SGG = 5
SG_EDGES = SGG * GROUP
SG_PROWS = SG_EDGES // PACK
N_SG = N_GROUPS // SGG
SEG = N_PACKED
NC = 2
NS = 16
NW = NC * NS
SG_MAX = 16


def _node_proj_body(n_ref, wr_ref, ws_ref, gh_ref, pr_ref, ps_ref):
    n = n_ref[...]
    dn = (((1,), (1,)), ((), ()))
    gh = gh_ref[...]
    pr_ref[...] = lax.dot_general(n, wr_ref[...], dn, preferred_element_type=jnp.float32) + gh
    ps_ref[...] = lax.dot_general(n, ws_ref[...], dn, preferred_element_type=jnp.float32) + gh


def _node_proj(nodes, wr, ws, gh):
    blk = 2000
    grid = N_NODES // blk
    return pl.pallas_call(
        _node_proj_body,
        grid=(grid,),
        in_specs=[
            pl.BlockSpec((blk, D_NODE), lambda i: (i, 0)),
            pl.BlockSpec((D_EDGE, D_NODE), lambda i: (0, 0)),
            pl.BlockSpec((D_EDGE, D_NODE), lambda i: (0, 0)),
            pl.BlockSpec((1, D_EDGE), lambda i: (0, 0)),
        ],
        out_specs=[
            pl.BlockSpec((blk, D_EDGE), lambda i: (i, 0)),
            pl.BlockSpec((blk, D_EDGE), lambda i: (i, 0)),
        ],
        out_shape=[
            jax.ShapeDtypeStruct((N_NODES, D_EDGE), jnp.float32),
            jax.ShapeDtypeStruct((N_NODES, D_EDGE), jnp.float32),
        ],
    )(nodes, wr, ws, gh)


SEGP = 40960
_EB = 1024


def _edge_linear_body(pe_ref, wbig_ref, o_ref):
    o_ref[...] = lax.dot_general(
        pe_ref[...], wbig_ref[...], (((1,), (0,)), ((), ())),
        preferred_element_type=jnp.float32)


def _edge_linear(pe_p, wbig):
    nsteps = SEGP // _EB
    return pl.pallas_call(
        _edge_linear_body,
        grid=(nsteps,),
        in_specs=[
            pl.BlockSpec((_EB, 128), lambda i: (i, 0)),
            pl.BlockSpec((128, 128), lambda i: (0, 0)),
        ],
        out_specs=pl.BlockSpec((_EB, 128), lambda i: (i, 0)),
        out_shape=jax.ShapeDtypeStruct((SEGP, 128), jnp.float32),
    )(pe_p, wbig)


def _sc_body(recv2, send2, pr, ps, e2, out2,
             idxr2, idxs2, rowr2, rows2, ebuf2, acc2,
             sem_ir0, sem_ir1, sem_is0, sem_is1,
             sem_gr0, sem_gr1, sem_gs0, sem_gs1,
             sem_e0, sem_e1, sem_st0, sem_st1):
    sem_ir = (sem_ir0, sem_ir1)
    sem_is = (sem_is0, sem_is1)
    sem_gr = (sem_gr0, sem_gr1)
    sem_gs = (sem_gs0, sem_gs1)
    sem_e = (sem_e0, sem_e1)
    sem_st = (sem_st0, sem_st1)

    c = lax.axis_index("c")
    s = lax.axis_index("s")
    wid = s * NC + c
    big = wid < 20
    n_sg = jnp.where(big, 16, 15)
    sg_base = jnp.where(big, wid * 16, 320 + (wid - 20) * 15)

    def sg_idx(i):
        return sg_base + jnp.minimum(i, n_sg - 1)

    def fire_idx(i, b):
        sg = sg_idx(i)
        ds_ = []
        for u in range(PACK):
            src = pl.ds(SEG * u + sg * SG_PROWS, SG_PROWS)
            dst = pl.ds(u * SG_PROWS, SG_PROWS)
            ds_.append(pltpu.async_copy(recv2.at[src], idxr2.at[b, dst], sem_ir[b]))
            ds_.append(pltpu.async_copy(send2.at[src], idxs2.at[b, dst], sem_is[b]))
        return ds_

    def fire_gathers(i, b):
        sg = sg_idx(i)
        ds_ = []
        for j in range(SGG):
            ds_.append(pltpu.async_copy(
                pr.at[idxr2.at[b, pl.ds(j * GROUP, GROUP)]],
                rowr2.at[b, pl.ds(j * GROUP, GROUP)], sem_gr[b]))
        for j in range(SGG):
            ds_.append(pltpu.async_copy(
                ps.at[idxs2.at[b, pl.ds(j * GROUP, GROUP)]],
                rows2.at[b, pl.ds(j * GROUP, GROUP)], sem_gs[b]))
        ds_.append(pltpu.async_copy(
            e2.at[pl.ds(sg * SG_PROWS, SG_PROWS)], ebuf2.at[b], sem_e[b]))
        return ds_

    def compute(b):
        def add_body(j, carry):
            for u in range(PACK):
                l = u * SG_PROWS + j
                acc2[b, j, pl.ds(u * D_EDGE, D_EDGE)] = (
                    ebuf2[b, j, pl.ds(u * D_EDGE, D_EDGE)]
                    + rowr2[b, l, :] + rows2[b, l, :])
            return carry
        lax.fori_loop(0, SG_PROWS, add_body, 0)

    for d in fire_idx(0, 0):
        d.wait()
    gat = [None, None]
    idxp = [None, None]
    stp = [None, None]
    gat[0] = fire_gathers(0, 0)
    idxp[1] = fire_idx(1, 1)

    for i in range(SG_MAX):
        b = i % 2
        nb = 1 - b
        for d in gat[b]:
            d.wait()
        if i < SG_MAX - 1:
            for d in idxp[nb]:
                d.wait()
            gat[nb] = fire_gathers(i + 1, nb)
            if i < SG_MAX - 2:
                idxp[b] = fire_idx(i + 2, b)
        if stp[b] is not None:
            stp[b].wait()
            stp[b] = None
        compute(b)
        stp[b] = pltpu.async_copy(
            acc2.at[b], out2.at[pl.ds(sg_idx(i) * SG_PROWS, SG_PROWS)], sem_st[b])

    for b in range(2):
        if stp[b] is not None:
            stp[b].wait()


@functools.partial(
    pl.kernel,
    mesh=plsc.VectorSubcoreMesh(core_axis_name="c", subcore_axis_name="s"),
    out_type=jax.ShapeDtypeStruct((SEGP, 128), jnp.float32),
    compiler_params=pltpu.CompilerParams(use_tc_tiling_on_sc=False),
    scratch_types=[
        pltpu.VMEM((2, SG_EDGES), jnp.int32),
        pltpu.VMEM((2, SG_EDGES), jnp.int32),
        pltpu.VMEM((2, SG_EDGES, D_EDGE), jnp.float32),
        pltpu.VMEM((2, SG_EDGES, D_EDGE), jnp.float32),
        pltpu.VMEM((2, SG_PROWS, 128), jnp.float32),
        pltpu.VMEM((2, SG_PROWS, 128), jnp.float32),
    ] + [pltpu.SemaphoreType.DMA] * 12,
)
def _sc_gather_add(recv2, send2, pr, ps, e2, out2, *scratch):
    _sc_body(recv2, send2, pr, ps, e2, out2, *scratch)


def kernel(nodes, edges, globals_, senders, receivers, W, b):
    we = W[:, :D_EDGE]
    wr = W[:, D_EDGE:D_EDGE + D_NODE]
    ws = W[:, D_EDGE + D_NODE:D_EDGE + 2 * D_NODE]
    wg = W[:, D_EDGE + 2 * D_NODE:]
    gvec = globals_ @ wg.T + b.reshape(1, D_EDGE)
    gh = 0.5 * gvec

    pe_p = jnp.pad(
        edges.reshape(PACK, SEG, D_EDGE), ((0, 0), (0, SEGP - SEG), (0, 0))
    ).transpose(1, 0, 2).reshape(SEGP, 128)
    wbig = jnp.kron(jnp.eye(PACK, dtype=jnp.float32), we.T)

    pr, ps = _node_proj(nodes, wr, ws, gh)
    e2p = _edge_linear(pe_p, wbig)

    out2p = _sc_gather_add(receivers, senders, pr, ps, e2p)
    return (out2p[:SEG].reshape(SEG, PACK, D_EDGE)
            .transpose(1, 0, 2).reshape(N_EDGES, D_EDGE))


# speedup vs baseline: 1.0082x; 1.0082x over previous
import functools

import jax
import jax.numpy as jnp
from jax import lax
from jax.experimental import pallas as pl
from jax.experimental.pallas import tpu as pltpu
from jax.experimental.pallas import tpu_sc as plsc

N_NODES = 10000
N_EDGES = 320000
D_NODE = 128
D_EDGE = 16
PACK = 128 // D_EDGE
N_PACKED = N_EDGES // PACK

GROUP = 128
N_GROUPS = N_EDGES // GROUP
SGG = 5
SG_EDGES = SGG * GROUP
SG_PROWS = SG_EDGES // PACK
N_SG = N_GROUPS // SGG
SEG = N_PACKED
NC = 2
NS = 16
NW = NC * NS
SG_MAX = 16


def _node_proj_body(n_ref, wr_ref, ws_ref, gh_ref, pr_ref, ps_ref):
    n = n_ref[...]
    dn = (((1,), (1,)), ((), ()))
    gh = gh_ref[...]
    pr_ref[...] = lax.dot_general(n, wr_ref[...], dn, preferred_element_type=jnp.float32) + gh
    ps_ref[...] = lax.dot_general(n, ws_ref[...], dn, preferred_element_type=jnp.float32) + gh


def _node_proj(nodes, wr, ws, gh):
    blk = 2000
    grid = N_NODES // blk
    return pl.pallas_call(
        _node_proj_body,
        grid=(grid,),
        in_specs=[
            pl.BlockSpec((blk, D_NODE), lambda i: (i, 0)),
            pl.BlockSpec((D_EDGE, D_NODE), lambda i: (0, 0)),
            pl.BlockSpec((D_EDGE, D_NODE), lambda i: (0, 0)),
            pl.BlockSpec((1, D_EDGE), lambda i: (0, 0)),
        ],
        out_specs=[
            pl.BlockSpec((blk, D_EDGE), lambda i: (i, 0)),
            pl.BlockSpec((blk, D_EDGE), lambda i: (i, 0)),
        ],
        out_shape=[
            jax.ShapeDtypeStruct((N_NODES, D_EDGE), jnp.float32),
            jax.ShapeDtypeStruct((N_NODES, D_EDGE), jnp.float32),
        ],
    )(nodes, wr, ws, gh)


SEGP = 40960
_EB = 1024


def _edge_linear_body(pe_ref, wbig_ref, o_ref):
    o_ref[...] = lax.dot_general(
        pe_ref[...], wbig_ref[...], (((1,), (0,)), ((), ())),
        preferred_element_type=jnp.float32)


def _edge_linear(pe_p, wbig):
    nsteps = SEGP // _EB
    return pl.pallas_call(
        _edge_linear_body,
        grid=(nsteps,),
        in_specs=[
            pl.BlockSpec((_EB, 128), lambda i: (i, 0)),
            pl.BlockSpec((128, 128), lambda i: (0, 0)),
        ],
        out_specs=pl.BlockSpec((_EB, 128), lambda i: (i, 0)),
        out_shape=jax.ShapeDtypeStruct((SEGP, 128), jnp.float32),
    )(pe_p, wbig)


def _sc_body(recv2, send2, pr, ps, e2, out2,
             idxr2, idxs2, rowr2, rows2, ebuf2, acc2,
             sem_ir0, sem_ir1, sem_is0, sem_is1,
             sem_gr0, sem_gr1, sem_gs0, sem_gs1,
             sem_e0, sem_e1, sem_st0, sem_st1):
    sem_ir = (sem_ir0, sem_ir1)
    sem_is = (sem_is0, sem_is1)
    sem_gr = (sem_gr0, sem_gr1)
    sem_gs = (sem_gs0, sem_gs1)
    sem_e = (sem_e0, sem_e1)
    sem_st = (sem_st0, sem_st1)

    c = lax.axis_index("c")
    s = lax.axis_index("s")
    wid = s * NC + c
    big = wid < 20
    n_sg = jnp.where(big, 16, 15)
    sg_base = jnp.where(big, wid * 16, 320 + (wid - 20) * 15)

    def sg_idx(i):
        return sg_base + jnp.minimum(i, n_sg - 1)

    def fire_idx(i, b):
        sg = sg_idx(i)
        ds_ = []
        for u in range(PACK):
            src = pl.ds(SEG * u + sg * SG_PROWS, SG_PROWS)
            dst = pl.ds(u * SG_PROWS, SG_PROWS)
            ds_.append(pltpu.async_copy(recv2.at[src], idxr2.at[b, dst], sem_ir[b]))
            ds_.append(pltpu.async_copy(send2.at[src], idxs2.at[b, dst], sem_is[b]))
        return ds_

    def fire_gathers(i, b):
        sg = sg_idx(i)
        ds_ = []
        for j in range(SGG):
            ds_.append(pltpu.async_copy(
                pr.at[idxr2.at[b, pl.ds(j * GROUP, GROUP)]],
                rowr2.at[b, pl.ds(j * GROUP, GROUP)], sem_gr[b]))
        for j in range(SGG):
            ds_.append(pltpu.async_copy(
                ps.at[idxs2.at[b, pl.ds(j * GROUP, GROUP)]],
                rows2.at[b, pl.ds(j * GROUP, GROUP)], sem_gs[b]))
        ds_.append(pltpu.async_copy(
            e2.at[pl.ds(sg * SG_PROWS, SG_PROWS)], ebuf2.at[b], sem_e[b]))
        return ds_

    def compute(b):
        def add_body(j, carry):
            for u in range(PACK):
                l = u * SG_PROWS + j
                acc2[b, j, pl.ds(u * D_EDGE, D_EDGE)] = (
                    ebuf2[b, j, pl.ds(u * D_EDGE, D_EDGE)]
                    + rowr2[b, l, :] + rows2[b, l, :])
            return carry
        lax.fori_loop(0, SG_PROWS, add_body, 0)

    for d in fire_idx(0, 0):
        d.wait()
    gat = [None, None]
    idxp = [None, None]
    stp = [None, None]
    gat[0] = fire_gathers(0, 0)
    idxp[1] = fire_idx(1, 1)

    for i in range(SG_MAX):
        b = i % 2
        nb = 1 - b
        for d in gat[b]:
            d.wait()
        if i < SG_MAX - 1:
            for d in idxp[nb]:
                d.wait()
            gat[nb] = fire_gathers(i + 1, nb)
            if i < SG_MAX - 2:
                idxp[b] = fire_idx(i + 2, b)
        if stp[b] is not None:
            stp[b].wait()
            stp[b] = None
        compute(b)
        stp[b] = pltpu.async_copy(
            acc2.at[b], out2.at[pl.ds(sg_idx(i) * SG_PROWS, SG_PROWS)], sem_st[b])

    for b in range(2):
        if stp[b] is not None:
            stp[b].wait()


@functools.partial(
    pl.kernel,
    mesh=plsc.VectorSubcoreMesh(core_axis_name="c", subcore_axis_name="s"),
    out_type=jax.ShapeDtypeStruct((SEGP, 128), jnp.float32),
    compiler_params=pltpu.CompilerParams(use_tc_tiling_on_sc=False),
    scratch_types=[
        pltpu.VMEM((2, SG_EDGES), jnp.int32),
        pltpu.VMEM((2, SG_EDGES), jnp.int32),
        pltpu.VMEM((2, SG_EDGES, D_EDGE), jnp.float32),
        pltpu.VMEM((2, SG_EDGES, D_EDGE), jnp.float32),
        pltpu.VMEM((2, SG_PROWS, 128), jnp.float32),
        pltpu.VMEM((2, SG_PROWS, 128), jnp.float32),
    ] + [pltpu.SemaphoreType.DMA] * 12,
)
def _sc_gather_add(recv2, send2, pr, ps, e2, out2, *scratch):
    _sc_body(recv2, send2, pr, ps, e2, out2, *scratch)


def kernel(nodes, edges, globals_, senders, receivers, W, b):
    we = W[:, :D_EDGE]
    wr = W[:, D_EDGE:D_EDGE + D_NODE]
    ws = W[:, D_EDGE + D_NODE:D_EDGE + 2 * D_NODE]
    wg = W[:, D_EDGE + 2 * D_NODE:]
    gvec = globals_ @ wg.T + b.reshape(1, D_EDGE)
    gh = 0.5 * gvec

    pe_p = jnp.pad(
        edges.reshape(PACK, SEG, D_EDGE).transpose(1, 0, 2).reshape(SEG, 128),
        ((0, SEGP - SEG), (0, 0)))
    wbig = jnp.kron(jnp.eye(PACK, dtype=jnp.float32), we.T)

    pr, ps = _node_proj(nodes, wr, ws, gh)
    e2p = _edge_linear(pe_p, wbig)

    out2p = _sc_gather_add(receivers, senders, pr, ps, e2p)
    return (out2p[:SEG].reshape(SEG, PACK, D_EDGE)
            .transpose(1, 0, 2).reshape(N_EDGES, D_EDGE))
